# Initial kernel scaffold; baseline (speedup 1.0000x reference)
import jax
import jax.numpy as jnp
from jax.experimental import pallas as pl


def kernel(x, edge_index, edge_attr, adj, W1s, W1n, b1, Wp1s, Wp1n, bp1, Wp2s, Wp2n, bp2, fc1_w, fc1_b, fc2_w, fc2_b):
    raise NotImplementedError("write your pallas kernel here")


# speedup vs baseline: 4.0306x; 4.0306x over previous
import functools

import jax
import jax.numpy as jnp
from jax import lax
from jax.experimental import pallas as pl
from jax.experimental.pallas import tpu as pltpu
from jax.experimental.pallas import tpu_sc as plsc

N = 10000
E = 160000
D = 128
W = 32
D1 = 48
NP = 10240
NC = 2
NS = 16
NW = NC * NS
CH = 128
EPW = 5120
EPAD = NW * EPW
RPS = NP // NS

BR = 200


def _k1_body(x_ref, ws_ref, wn_ref, cs_ref, t_ref):
    x = x_ref[...]
    cs_ref[...] = jnp.dot(x, ws_ref[...], preferred_element_type=jnp.float32)
    t = jnp.dot(x, wn_ref[...], preferred_element_type=jnp.float32)
    col = lax.broadcasted_iota(jnp.int32, t.shape, 1)
    t_ref[...] = jnp.where(col == W, 1.0, t)


def _k1_call(x, ws_p, wn_p):
    blk = 1000
    return pl.pallas_call(
        _k1_body,
        grid=(N // blk,),
        in_specs=[
            pl.BlockSpec((blk, D), lambda i: (i, 0)),
            pl.BlockSpec((D, W), lambda i: (0, 0)),
            pl.BlockSpec((D, D1), lambda i: (0, 0)),
        ],
        out_specs=[
            pl.BlockSpec((blk, W), lambda i: (i, 0)),
            pl.BlockSpec((blk, D1), lambda i: (i, 0)),
        ],
        out_shape=[
            jax.ShapeDtypeStruct((N, W), jnp.float32),
            jax.ShapeDtypeStruct((N, D1), jnp.float32),
        ],
    )(x, ws_p, wn_p)


def _make_prop(dt):
    mesh = plsc.VectorSubcoreMesh(core_axis_name="c", subcore_axis_name="s",
                                  num_cores=NC, num_subcores=NS)

    @functools.partial(
        pl.kernel,
        out_type=jax.ShapeDtypeStruct((NC * NP, dt), jnp.float32),
        mesh=mesh,
        compiler_params=pltpu.CompilerParams(use_tc_tiling_on_sc=False),
        scratch_types=[
            pltpu.VMEM((CH,), jnp.int32),
            pltpu.VMEM((CH,), jnp.int32),
            pltpu.VMEM((CH, dt), jnp.float32),
            pltpu.VMEM_SHARED((NP, dt), jnp.float32),
            pltpu.SemaphoreType.DMA,
        ],
    )
    def prop(table_hbm, src_hbm, dst_hbm, zeros_hbm, out_hbm,
             sidx, didx, rows, acc, sem):
        c = lax.axis_index("c")
        s = lax.axis_index("s")
        wid = s * NC + c
        pltpu.sync_copy(zeros_hbm, acc.at[pl.ds(s * RPS, RPS)])
        plsc.subcore_barrier()

        base = wid * EPW

        def body(ci, carry):
            off = base + ci * CH
            pltpu.sync_copy(src_hbm.at[pl.ds(off, CH)], sidx)
            pltpu.async_copy(table_hbm.at[sidx], rows, sem).wait()
            pltpu.sync_copy(dst_hbm.at[pl.ds(off, CH)], didx)
            pltpu.sync_copy(rows, acc.at[didx], add=True)
            return carry

        lax.fori_loop(0, EPW // CH, body, 0)
        plsc.subcore_barrier()
        pltpu.sync_copy(acc.at[pl.ds(s * RPS, RPS)],
                        out_hbm.at[pl.ds(c * NP + s * RPS, RPS)])

    return prop


_prop_cache = {}


def _get_prop(dt):
    if dt not in _prop_cache:
        _prop_cache[dt] = _make_prop(dt)
    return _prop_cache[dt]


def _k3_body(cs_ref, a1_ref, a2_ref, b1_ref, h_ref, inv_ref):
    a = a1_ref[...] + a2_ref[...]
    deg = a[:, W:W + 1]
    inv = 1.0 / jnp.maximum(deg, 1.0)
    h_ref[...] = cs_ref[...] + a[:, :W] * inv + b1_ref[...]
    inv_ref[...] = inv


def _k3_call(cs, a1, a2, b1_p):
    blk = 1000
    return pl.pallas_call(
        _k3_body,
        grid=(N // blk,),
        in_specs=[
            pl.BlockSpec((blk, W), lambda i: (i, 0)),
            pl.BlockSpec((blk, D1), lambda i: (i, 0)),
            pl.BlockSpec((blk, D1), lambda i: (i, 0)),
            pl.BlockSpec((1, W), lambda i: (0, 0)),
        ],
        out_specs=[
            pl.BlockSpec((blk, W), lambda i: (i, 0)),
            pl.BlockSpec((blk, 1), lambda i: (i, 0)),
        ],
        out_shape=[
            jax.ShapeDtypeStruct((N, W), jnp.float32),
            jax.ShapeDtypeStruct((N, 1), jnp.float32),
        ],
    )(cs, a1, a2, b1_p)


def _k5_body(h_ref, a1_ref, a2_ref, inv_ref, ws_ref, wn_ref, b_ref, adj_ref,
             p_ref, y_ref, ssq_ref):
    i = pl.program_id(0)
    h = h_ref[...]
    aggm = (a1_ref[...] + a2_ref[...]) * inv_ref[...]
    s1 = (jnp.dot(h, ws_ref[...], preferred_element_type=jnp.float32)
          + jnp.dot(aggm, wn_ref[...], preferred_element_type=jnp.float32)
          + b_ref[...])
    m = jnp.max(s1, axis=-1, keepdims=True)
    e = jnp.exp(s1 - m)
    p = e / jnp.sum(e, axis=-1, keepdims=True)
    p_ref[...] = p

    adj = adj_ref[...]

    @pl.when(i == 0)
    def _init():
        y_ref[...] = jnp.zeros_like(y_ref)
        ssq_ref[0, 0] = 0.0

    y_ref[...] += lax.dot_general(p, adj, (((0,), (0,)), ((), ())),
                                  preferred_element_type=jnp.float32)
    ssq_ref[0, 0] += jnp.sum(adj * adj)


def _k5_call(h, a1, a2, inv, ws_p, wn_p, b_p, adj):
    return pl.pallas_call(
        _k5_body,
        grid=(N // BR,),
        in_specs=[
            pl.BlockSpec((BR, W), lambda i: (i, 0)),
            pl.BlockSpec((BR, W), lambda i: (i, 0)),
            pl.BlockSpec((BR, W), lambda i: (i, 0)),
            pl.BlockSpec((BR, 1), lambda i: (i, 0)),
            pl.BlockSpec((W, W), lambda i: (0, 0)),
            pl.BlockSpec((W, W), lambda i: (0, 0)),
            pl.BlockSpec((1, W), lambda i: (0, 0)),
            pl.BlockSpec((BR, N), lambda i: (i, 0)),
        ],
        out_specs=[
            pl.BlockSpec((BR, W), lambda i: (i, 0)),
            pl.BlockSpec((W, N), lambda i: (0, 0)),
            pl.BlockSpec((1, 1), lambda i: (0, 0), memory_space=pltpu.SMEM),
        ],
        out_shape=[
            jax.ShapeDtypeStruct((N, W), jnp.float32),
            jax.ShapeDtypeStruct((W, N), jnp.float32),
            jax.ShapeDtypeStruct((1, 1), jnp.float32),
        ],
    )(h, a1, a2, inv, ws_p, wn_p, b_p, adj)


def _k6_body(y_ref, p_ref, h_ref, ssq_ref, w2s_ref, w2n_ref, b2_ref,
             fc1w_ref, fc1b_ref, fc2w_ref, fc2b_ref, z_ref, reg_ref):
    y = y_ref[...]
    p = p_ref[...]
    h = h_ref[...]

    adj1 = jnp.dot(y, p, preferred_element_type=jnp.float32)
    sts = lax.dot_general(p, p, (((0,), (0,)), ((), ())),
                          preferred_element_type=jnp.float32)
    h1 = lax.dot_general(p, h, (((0,), (0,)), ((), ())),
                         preferred_element_type=jnp.float32)

    rid = lax.broadcasted_iota(jnp.int32, (W, W), 0)
    cid = lax.broadcasted_iota(jnp.int32, (W, W), 1)
    tr = jnp.sum(jnp.where(rid == cid, adj1, 0.0))
    ssq = ssq_ref[0, 0]
    l1 = jnp.sqrt(jnp.maximum(ssq - 2.0 * tr + jnp.sum(sts * sts), 0.0))
    link1 = l1 / (N * N)
    ent1 = jnp.sum(-p * jnp.log(p + 1e-15)) / N
    reg1 = link1 + ent1

    m1 = jnp.sum(h1, axis=0, keepdims=True) / W
    s2 = (jnp.dot(h1, w2s_ref[...], preferred_element_type=jnp.float32)
          + jnp.dot(m1, w2n_ref[...], preferred_element_type=jnp.float32)
          + b2_ref[...])
    s2 = jnp.where(cid < 4, s2, -1e30)
    mx = jnp.max(s2, axis=-1, keepdims=True)
    e2 = jnp.exp(s2 - mx)
    p2 = e2 / jnp.sum(e2, axis=-1, keepdims=True)

    h2 = lax.dot_general(p2, h1, (((0,), (0,)), ((), ())),
                         preferred_element_type=jnp.float32)
    pp2 = lax.dot_general(p2, p2, (((1,), (1,)), ((), ())),
                          preferred_element_type=jnp.float32)
    dif = adj1 - pp2
    link2 = jnp.sqrt(jnp.sum(dif * dif)) / (W * W)
    ent2 = jnp.sum(-p2 * jnp.log(p2 + 1e-15)) / W
    reg2 = link2 + ent2

    z = jnp.zeros((1, W), jnp.float32)
    for r in range(4):
        row = jnp.sum(jnp.where(rid == r, h2, 0.0), axis=0, keepdims=True)
        z = z + jnp.dot(row, fc1w_ref[r], preferred_element_type=jnp.float32)
    z = jnp.maximum(z + fc1b_ref[...], 0.0)
    z_ref[...] = jnp.dot(z, fc2w_ref[...],
                         preferred_element_type=jnp.float32) + fc2b_ref[...]
    reg_ref[0, 0] = reg1 * 10.0 + reg2 * 0.1


def _k6_call(y, p, h, ssq, w2s_p, w2n_p, b2_p, fc1w_p, fc1b_p, fc2w, fc2b_p):
    return pl.pallas_call(
        _k6_body,
        in_specs=[
            pl.BlockSpec((W, N), lambda: (0, 0)),
            pl.BlockSpec((N, W), lambda: (0, 0)),
            pl.BlockSpec((N, W), lambda: (0, 0)),
            pl.BlockSpec((1, 1), lambda: (0, 0), memory_space=pltpu.SMEM),
            pl.BlockSpec((W, W), lambda: (0, 0)),
            pl.BlockSpec((W, W), lambda: (0, 0)),
            pl.BlockSpec((1, W), lambda: (0, 0)),
            pl.BlockSpec((4, W, W), lambda: (0, 0, 0)),
            pl.BlockSpec((1, W), lambda: (0, 0)),
            pl.BlockSpec((W, 2), lambda: (0, 0)),
            pl.BlockSpec((1, 2), lambda: (0, 0)),
        ],
        out_specs=[
            pl.BlockSpec((1, 2), lambda: (0, 0)),
            pl.BlockSpec((1, 1), lambda: (0, 0), memory_space=pltpu.SMEM),
        ],
        out_shape=[
            jax.ShapeDtypeStruct((1, 2), jnp.float32),
            jax.ShapeDtypeStruct((1, 1), jnp.float32),
        ],
    )(y, p, h, ssq, w2s_p, w2n_p, b2_p, fc1w_p, fc1b_p, fc2w, fc2b_p)


def kernel(x, edge_index, edge_attr, adj, W1s, W1n, b1, Wp1s, Wp1n, bp1,
           Wp2s, Wp2n, bp2, fc1_w, fc1_b, fc2_w, fc2_b):
    f32 = jnp.float32

    ws_p = jnp.zeros((D, W), f32).at[:, :30].set(W1s)
    wn_p = jnp.zeros((D, D1), f32).at[:, :30].set(W1n)
    b1_p = jnp.zeros((1, W), f32).at[0, :30].set(b1)
    wp1s_p = jnp.zeros((W, W), f32).at[:30, :].set(Wp1s)
    wp1n_p = jnp.zeros((W, W), f32).at[:30, :].set(Wp1n)
    bp1_p = bp1.reshape(1, W)
    wp2s_p = jnp.zeros((W, W), f32).at[:30, :4].set(Wp2s)
    wp2n_p = jnp.zeros((W, W), f32).at[:30, :4].set(Wp2n)
    bp2_p = jnp.zeros((1, W), f32).at[0, :4].set(bp2)
    fc1w_p = jnp.zeros((4, W, W), f32).at[:, :30, :].set(
        fc1_w.reshape(4, 30, W))
    fc1b_p = fc1_b.reshape(1, W)
    fc2b_p = fc2_b.reshape(1, 2)

    src_p = jnp.concatenate(
        [edge_index[0], jnp.zeros((EPAD - E,), jnp.int32)])
    dst_p = jnp.concatenate(
        [edge_index[1], jnp.full((EPAD - E,), N + 200, jnp.int32)])
    zeros48 = jnp.zeros((RPS, D1), f32)
    zeros32 = jnp.zeros((RPS, W), f32)

    cs, table1 = _k1_call(x, ws_p, wn_p)

    agg1 = _get_prop(D1)(table1, src_p, dst_p, zeros48)
    a1a = lax.slice(agg1, (0, 0), (N, D1))
    a1b = lax.slice(agg1, (NP, 0), (NP + N, D1))

    h, invdeg = _k3_call(cs, a1a, a1b, b1_p)

    agg2 = _get_prop(W)(h, src_p, dst_p, zeros32)
    a2a = lax.slice(agg2, (0, 0), (N, W))
    a2b = lax.slice(agg2, (NP, 0), (NP + N, W))

    p, y, ssq = _k5_call(h, a2a, a2b, invdeg, wp1s_p, wp1n_p, bp1_p, adj)

    z, reg = _k6_call(y, p, h, ssq, wp2s_p, wp2n_p, bp2_p, fc1w_p, fc1b_p,
                      fc2_w, fc2b_p)
    return z, reg[0, 0]


# speedup vs baseline: 5.2017x; 1.2906x over previous
import functools

import jax
import jax.numpy as jnp
from jax import lax
from jax.experimental import pallas as pl
from jax.experimental.pallas import tpu as pltpu
from jax.experimental.pallas import tpu_sc as plsc

N = 10000
E = 160000
D = 128
W = 32
NP = 10240
NC = 2
NS = 16
NW = NC * NS
CH = 128
EPW = 5120
EPAD = NW * EPW
RPS = NP // NS
NCHK = EPW // CH
GK = 8
NG = NCHK // GK

BR = 200


def _k1_body(x_ref, ws_ref, wn_ref, cs_ref, t_ref):
    x = x_ref[...]
    cs_ref[...] = jnp.dot(x, ws_ref[...], preferred_element_type=jnp.float32)
    t = jnp.dot(x, wn_ref[...], preferred_element_type=jnp.float32)
    col = lax.broadcasted_iota(jnp.int32, t.shape, 1)
    t_ref[...] = jnp.where(col == W - 1, 1.0, t)


def _k1_call(x, ws_p, wn_p):
    blk = 1000
    return pl.pallas_call(
        _k1_body,
        grid=(N // blk,),
        in_specs=[
            pl.BlockSpec((blk, D), lambda i: (i, 0)),
            pl.BlockSpec((D, W), lambda i: (0, 0)),
            pl.BlockSpec((D, W), lambda i: (0, 0)),
        ],
        out_specs=[
            pl.BlockSpec((blk, W), lambda i: (i, 0)),
            pl.BlockSpec((blk, W), lambda i: (i, 0)),
        ],
        out_shape=[
            jax.ShapeDtypeStruct((N, W), jnp.float32),
            jax.ShapeDtypeStruct((N, W), jnp.float32),
        ],
    )(x, ws_p, wn_p)


def _make_prop(dt):
    mesh = plsc.VectorSubcoreMesh(core_axis_name="c", subcore_axis_name="s",
                                  num_cores=NC, num_subcores=NS)

    @functools.partial(
        pl.kernel,
        out_type=jax.ShapeDtypeStruct((NC * NP, dt), jnp.float32),
        mesh=mesh,
        compiler_params=pltpu.CompilerParams(use_tc_tiling_on_sc=False),
        scratch_types=[
            pltpu.VMEM((NCHK, CH), jnp.int32),
            pltpu.VMEM((NCHK, CH), jnp.int32),
            pltpu.VMEM((GK, CH, dt), jnp.float32),
            pltpu.VMEM_SHARED((NP, dt), jnp.float32),
            pltpu.SemaphoreType.DMA,
        ],
    )
    def prop(table_hbm, src_hbm, dst_hbm, zeros_hbm, out_hbm,
             sidx, didx, rows, acc, sem):
        c = lax.axis_index("c")
        s = lax.axis_index("s")
        wid = s * NC + c
        pltpu.sync_copy(zeros_hbm, acc.at[pl.ds(s * RPS, RPS)])
        pltpu.sync_copy(src_hbm.at[pl.ds(wid * NCHK, NCHK)], sidx)
        pltpu.sync_copy(dst_hbm.at[pl.ds(wid * NCHK, NCHK)], didx)
        plsc.subcore_barrier()

        def group(g, carry):
            base = g * GK
            descs = [
                pltpu.async_copy(table_hbm.at[sidx.at[base + b]],
                                 rows.at[b], sem)
                for b in range(GK)
            ]
            for d in descs:
                d.wait()
            for b in range(GK):
                pltpu.sync_copy(rows.at[b], acc.at[didx.at[base + b]],
                                add=True)
            return carry

        lax.fori_loop(0, NG, group, 0)
        plsc.subcore_barrier()
        pltpu.sync_copy(acc.at[pl.ds(s * RPS, RPS)],
                        out_hbm.at[pl.ds(c * NP + s * RPS, RPS)])

    return prop


_prop_cache = {}


def _get_prop(dt):
    if dt not in _prop_cache:
        _prop_cache[dt] = _make_prop(dt)
    return _prop_cache[dt]


def _k3_body(cs_ref, a1_ref, a2_ref, b1_ref, h_ref, inv_ref):
    a = a1_ref[...] + a2_ref[...]
    deg = a[:, W - 1:W]
    inv = 1.0 / jnp.maximum(deg, 1.0)
    h = cs_ref[...] + a * inv + b1_ref[...]
    col = lax.broadcasted_iota(jnp.int32, h.shape, 1)
    h_ref[...] = jnp.where(col == W - 1, 0.0, h)
    inv_ref[...] = inv


def _k3_call(cs, a1, a2, b1_p):
    blk = 1000
    return pl.pallas_call(
        _k3_body,
        grid=(N // blk,),
        in_specs=[
            pl.BlockSpec((blk, W), lambda i: (i, 0)),
            pl.BlockSpec((blk, W), lambda i: (i, 0)),
            pl.BlockSpec((blk, W), lambda i: (i, 0)),
            pl.BlockSpec((1, W), lambda i: (0, 0)),
        ],
        out_specs=[
            pl.BlockSpec((blk, W), lambda i: (i, 0)),
            pl.BlockSpec((blk, 1), lambda i: (i, 0)),
        ],
        out_shape=[
            jax.ShapeDtypeStruct((N, W), jnp.float32),
            jax.ShapeDtypeStruct((N, 1), jnp.float32),
        ],
    )(cs, a1, a2, b1_p)


def _k5_body(h_ref, a1_ref, a2_ref, inv_ref, ws_ref, wn_ref, b_ref, adj_ref,
             p_ref, y_ref, ssq_ref):
    i = pl.program_id(0)
    h = h_ref[...]
    aggm = (a1_ref[...] + a2_ref[...]) * inv_ref[...]
    s1 = (jnp.dot(h, ws_ref[...], preferred_element_type=jnp.float32)
          + jnp.dot(aggm, wn_ref[...], preferred_element_type=jnp.float32)
          + b_ref[...])
    m = jnp.max(s1, axis=-1, keepdims=True)
    e = jnp.exp(s1 - m)
    p = e / jnp.sum(e, axis=-1, keepdims=True)
    p_ref[...] = p

    adj = adj_ref[...]

    @pl.when(i == 0)
    def _init():
        y_ref[...] = jnp.zeros_like(y_ref)
        ssq_ref[0, 0] = 0.0

    y_ref[...] += lax.dot_general(p, adj, (((0,), (0,)), ((), ())),
                                  preferred_element_type=jnp.float32)
    ssq_ref[0, 0] += jnp.sum(adj * adj)


def _k5_call(h, a1, a2, inv, ws_p, wn_p, b_p, adj):
    return pl.pallas_call(
        _k5_body,
        grid=(N // BR,),
        in_specs=[
            pl.BlockSpec((BR, W), lambda i: (i, 0)),
            pl.BlockSpec((BR, W), lambda i: (i, 0)),
            pl.BlockSpec((BR, W), lambda i: (i, 0)),
            pl.BlockSpec((BR, 1), lambda i: (i, 0)),
            pl.BlockSpec((W, W), lambda i: (0, 0)),
            pl.BlockSpec((W, W), lambda i: (0, 0)),
            pl.BlockSpec((1, W), lambda i: (0, 0)),
            pl.BlockSpec((BR, N), lambda i: (i, 0)),
        ],
        out_specs=[
            pl.BlockSpec((BR, W), lambda i: (i, 0)),
            pl.BlockSpec((W, N), lambda i: (0, 0)),
            pl.BlockSpec((1, 1), lambda i: (0, 0), memory_space=pltpu.SMEM),
        ],
        out_shape=[
            jax.ShapeDtypeStruct((N, W), jnp.float32),
            jax.ShapeDtypeStruct((W, N), jnp.float32),
            jax.ShapeDtypeStruct((1, 1), jnp.float32),
        ],
    )(h, a1, a2, inv, ws_p, wn_p, b_p, adj)


def _k6_body(y_ref, p_ref, h_ref, ssq_ref, w2s_ref, w2n_ref, b2_ref,
             fc1w_ref, fc1b_ref, fc2w_ref, fc2b_ref, z_ref, reg_ref):
    y = y_ref[...]
    p = p_ref[...]
    h = h_ref[...]

    adj1 = jnp.dot(y, p, preferred_element_type=jnp.float32)
    sts = lax.dot_general(p, p, (((0,), (0,)), ((), ())),
                          preferred_element_type=jnp.float32)
    h1 = lax.dot_general(p, h, (((0,), (0,)), ((), ())),
                         preferred_element_type=jnp.float32)

    rid = lax.broadcasted_iota(jnp.int32, (W, W), 0)
    cid = lax.broadcasted_iota(jnp.int32, (W, W), 1)
    tr = jnp.sum(jnp.where(rid == cid, adj1, 0.0))
    ssq = ssq_ref[0, 0]
    l1 = jnp.sqrt(jnp.maximum(ssq - 2.0 * tr + jnp.sum(sts * sts), 0.0))
    link1 = l1 / (N * N)
    ent1 = jnp.sum(-p * jnp.log(p + 1e-15)) / N
    reg1 = link1 + ent1

    m1 = jnp.sum(h1, axis=0, keepdims=True) / W
    s2 = (jnp.dot(h1, w2s_ref[...], preferred_element_type=jnp.float32)
          + jnp.dot(m1, w2n_ref[...], preferred_element_type=jnp.float32)
          + b2_ref[...])
    s2 = jnp.where(cid < 4, s2, -1e30)
    mx = jnp.max(s2, axis=-1, keepdims=True)
    e2 = jnp.exp(s2 - mx)
    p2 = e2 / jnp.sum(e2, axis=-1, keepdims=True)

    h2 = lax.dot_general(p2, h1, (((0,), (0,)), ((), ())),
                         preferred_element_type=jnp.float32)
    pp2 = lax.dot_general(p2, p2, (((1,), (1,)), ((), ())),
                          preferred_element_type=jnp.float32)
    dif = adj1 - pp2
    link2 = jnp.sqrt(jnp.sum(dif * dif)) / (W * W)
    ent2 = jnp.sum(-p2 * jnp.log(p2 + 1e-15)) / W
    reg2 = link2 + ent2

    z = jnp.zeros((1, W), jnp.float32)
    for r in range(4):
        row = jnp.sum(jnp.where(rid == r, h2, 0.0), axis=0, keepdims=True)
        z = z + jnp.dot(row, fc1w_ref[r], preferred_element_type=jnp.float32)
    z = jnp.maximum(z + fc1b_ref[...], 0.0)
    z_ref[...] = jnp.dot(z, fc2w_ref[...],
                         preferred_element_type=jnp.float32) + fc2b_ref[...]
    reg_ref[0, 0] = reg1 * 10.0 + reg2 * 0.1


def _k6_call(y, p, h, ssq, w2s_p, w2n_p, b2_p, fc1w_p, fc1b_p, fc2w, fc2b_p):
    return pl.pallas_call(
        _k6_body,
        in_specs=[
            pl.BlockSpec((W, N), lambda: (0, 0)),
            pl.BlockSpec((N, W), lambda: (0, 0)),
            pl.BlockSpec((N, W), lambda: (0, 0)),
            pl.BlockSpec((1, 1), lambda: (0, 0), memory_space=pltpu.SMEM),
            pl.BlockSpec((W, W), lambda: (0, 0)),
            pl.BlockSpec((W, W), lambda: (0, 0)),
            pl.BlockSpec((1, W), lambda: (0, 0)),
            pl.BlockSpec((4, W, W), lambda: (0, 0, 0)),
            pl.BlockSpec((1, W), lambda: (0, 0)),
            pl.BlockSpec((W, 2), lambda: (0, 0)),
            pl.BlockSpec((1, 2), lambda: (0, 0)),
        ],
        out_specs=[
            pl.BlockSpec((1, 2), lambda: (0, 0)),
            pl.BlockSpec((1, 1), lambda: (0, 0), memory_space=pltpu.SMEM),
        ],
        out_shape=[
            jax.ShapeDtypeStruct((1, 2), jnp.float32),
            jax.ShapeDtypeStruct((1, 1), jnp.float32),
        ],
    )(y, p, h, ssq, w2s_p, w2n_p, b2_p, fc1w_p, fc1b_p, fc2w, fc2b_p)


def kernel(x, edge_index, edge_attr, adj, W1s, W1n, b1, Wp1s, Wp1n, bp1,
           Wp2s, Wp2n, bp2, fc1_w, fc1_b, fc2_w, fc2_b):
    f32 = jnp.float32

    ws_p = jnp.zeros((D, W), f32).at[:, :30].set(W1s)
    wn_p = jnp.zeros((D, W), f32).at[:, :30].set(W1n)
    b1_p = jnp.zeros((1, W), f32).at[0, :30].set(b1)
    wp1s_p = jnp.zeros((W, W), f32).at[:30, :].set(Wp1s)
    wp1n_p = jnp.zeros((W, W), f32).at[:30, :].set(Wp1n)
    bp1_p = bp1.reshape(1, W)
    wp2s_p = jnp.zeros((W, W), f32).at[:30, :4].set(Wp2s)
    wp2n_p = jnp.zeros((W, W), f32).at[:30, :4].set(Wp2n)
    bp2_p = jnp.zeros((1, W), f32).at[0, :4].set(bp2)
    fc1w_p = jnp.zeros((4, W, W), f32).at[:, :30, :].set(
        fc1_w.reshape(4, 30, W))
    fc1b_p = fc1_b.reshape(1, W)
    fc2b_p = fc2_b.reshape(1, 2)

    src3 = jnp.concatenate(
        [edge_index[0], jnp.zeros((EPAD - E,), jnp.int32)]).reshape(
            NW * NCHK, CH)
    dst3 = jnp.concatenate(
        [edge_index[1], jnp.full((EPAD - E,), N + 200, jnp.int32)]).reshape(
            NW * NCHK, CH)
    zeros32 = jnp.zeros((RPS, W), f32)

    cs, table1 = _k1_call(x, ws_p, wn_p)

    agg1 = _get_prop(W)(table1, src3, dst3, zeros32)
    a1a = lax.slice(agg1, (0, 0), (N, W))
    a1b = lax.slice(agg1, (NP, 0), (NP + N, W))

    h, invdeg = _k3_call(cs, a1a, a1b, b1_p)

    agg2 = _get_prop(W)(h, src3, dst3, zeros32)
    a2a = lax.slice(agg2, (0, 0), (N, W))
    a2b = lax.slice(agg2, (NP, 0), (NP + N, W))

    p, y, ssq = _k5_call(h, a2a, a2b, invdeg, wp1s_p, wp1n_p, bp1_p, adj)

    z, reg = _k6_call(y, p, h, ssq, wp2s_p, wp2n_p, bp2_p, fc1w_p, fc1b_p,
                      fc2_w, fc2b_p)
    return z, reg[0, 0]


# speedup vs baseline: 5.4897x; 1.0554x over previous
import functools

import jax
import jax.numpy as jnp
from jax import lax
from jax.experimental import pallas as pl
from jax.experimental.pallas import tpu as pltpu
from jax.experimental.pallas import tpu_sc as plsc

N = 10000
E = 160000
D = 128
W = 32
NP = 10240
NC = 2
NS = 16
NW = NC * NS
CH = 128
EPW = 5120
EPAD = NW * EPW
RPS = NP // NS
NCHK = EPW // CH
GK = 8
NG = NCHK // GK

BR = 400


def _k1_body(x_ref, ws_ref, wn_ref, cs_ref, t_ref):
    x = x_ref[...]
    cs_ref[...] = jnp.dot(x, ws_ref[...], preferred_element_type=jnp.float32)
    t = jnp.dot(x, wn_ref[...], preferred_element_type=jnp.float32)
    col = lax.broadcasted_iota(jnp.int32, t.shape, 1)
    t_ref[...] = jnp.where(col == W - 1, 1.0, t)


def _k1_call(x, ws_p, wn_p):
    blk = 1000
    return pl.pallas_call(
        _k1_body,
        grid=(N // blk,),
        in_specs=[
            pl.BlockSpec((blk, D), lambda i: (i, 0)),
            pl.BlockSpec((D, W), lambda i: (0, 0)),
            pl.BlockSpec((D, W), lambda i: (0, 0)),
        ],
        out_specs=[
            pl.BlockSpec((blk, W), lambda i: (i, 0)),
            pl.BlockSpec((blk, W), lambda i: (i, 0)),
        ],
        out_shape=[
            jax.ShapeDtypeStruct((N, W), jnp.float32),
            jax.ShapeDtypeStruct((N, W), jnp.float32),
        ],
    )(x, ws_p, wn_p)


def _make_prop(dt):
    mesh = plsc.VectorSubcoreMesh(core_axis_name="c", subcore_axis_name="s",
                                  num_cores=NC, num_subcores=NS)

    @functools.partial(
        pl.kernel,
        out_type=jax.ShapeDtypeStruct((NC * NP, dt), jnp.float32),
        mesh=mesh,
        compiler_params=pltpu.CompilerParams(use_tc_tiling_on_sc=False),
        scratch_types=[
            pltpu.VMEM((NCHK, CH), jnp.int32),
            pltpu.VMEM((NCHK, CH), jnp.int32),
            pltpu.VMEM((GK, CH, dt), jnp.float32),
            pltpu.VMEM_SHARED((NP, dt), jnp.float32),
            pltpu.SemaphoreType.DMA,
            pltpu.SemaphoreType.DMA,
        ],
    )
    def prop(table_hbm, src_hbm, dst_hbm, zeros_hbm, out_hbm,
             sidx, didx, rows, acc, sem_g, sem_s):
        c = lax.axis_index("c")
        s = lax.axis_index("s")
        wid = s * NC + c
        pltpu.sync_copy(zeros_hbm, acc.at[pl.ds(s * RPS, RPS)])
        pltpu.sync_copy(src_hbm.at[pl.ds(wid * NCHK, NCHK)], sidx)
        pltpu.sync_copy(dst_hbm.at[pl.ds(wid * NCHK, NCHK)], didx)
        plsc.subcore_barrier()

        def drain_scatters():
            for b in range(GK):
                pltpu.make_async_copy(table_hbm.at[pl.ds(0, CH)],
                                      rows.at[b], sem_s).wait()

        def group(g, carry):
            @pl.when(g > 0)
            def _():
                drain_scatters()
            base = g * GK
            descs = [
                pltpu.async_copy(table_hbm.at[sidx.at[base + b]],
                                 rows.at[b], sem_g)
                for b in range(GK)
            ]
            for d in descs:
                d.wait()
            for b in range(GK):
                pltpu.async_copy(rows.at[b], acc.at[didx.at[base + b]],
                                 sem_s, add=True)
            return carry

        lax.fori_loop(0, NG, group, 0)
        drain_scatters()
        plsc.subcore_barrier()
        pltpu.sync_copy(acc.at[pl.ds(s * RPS, RPS)],
                        out_hbm.at[pl.ds(c * NP + s * RPS, RPS)])

    return prop


_prop_cache = {}


def _get_prop(dt):
    if dt not in _prop_cache:
        _prop_cache[dt] = _make_prop(dt)
    return _prop_cache[dt]


def _k3_body(cs_ref, a1_ref, a2_ref, b1_ref, h_ref, inv_ref):
    a = a1_ref[...] + a2_ref[...]
    deg = a[:, W - 1:W]
    inv = 1.0 / jnp.maximum(deg, 1.0)
    h = cs_ref[...] + a * inv + b1_ref[...]
    col = lax.broadcasted_iota(jnp.int32, h.shape, 1)
    h_ref[...] = jnp.where(col == W - 1, 0.0, h)
    inv_ref[...] = inv


def _k3_call(cs, a1, a2, b1_p):
    blk = 1000
    return pl.pallas_call(
        _k3_body,
        grid=(N // blk,),
        in_specs=[
            pl.BlockSpec((blk, W), lambda i: (i, 0)),
            pl.BlockSpec((blk, W), lambda i: (i, 0)),
            pl.BlockSpec((blk, W), lambda i: (i, 0)),
            pl.BlockSpec((1, W), lambda i: (0, 0)),
        ],
        out_specs=[
            pl.BlockSpec((blk, W), lambda i: (i, 0)),
            pl.BlockSpec((blk, 1), lambda i: (i, 0)),
        ],
        out_shape=[
            jax.ShapeDtypeStruct((N, W), jnp.float32),
            jax.ShapeDtypeStruct((N, 1), jnp.float32),
        ],
    )(cs, a1, a2, b1_p)


def _k5_body(h_ref, a1_ref, a2_ref, inv_ref, ws_ref, wn_ref, b_ref, adj_ref,
             w2s_ref, w2n_ref, b2_ref, fc1w_ref, fc1b_ref, fc2w_ref,
             fc2b_ref, z_ref, reg_ref, p_scr, y_scr, ssq_scr):
    i = pl.program_id(0)
    nb = pl.num_programs(0)
    h_blk = h_ref[pl.ds(i * BR, BR), :]
    aggm = (a1_ref[...] + a2_ref[...]) * inv_ref[...]
    s1 = (jnp.dot(h_blk, ws_ref[...], preferred_element_type=jnp.float32)
          + jnp.dot(aggm, wn_ref[...], preferred_element_type=jnp.float32)
          + b_ref[...])
    m = jnp.max(s1, axis=-1, keepdims=True)
    e = jnp.exp(s1 - m)
    p = e / jnp.sum(e, axis=-1, keepdims=True)
    p_scr[pl.ds(i * BR, BR), :] = p

    adj = adj_ref[...]

    @pl.when(i == 0)
    def _init():
        y_scr[...] = jnp.zeros_like(y_scr)
        ssq_scr[0, 0] = 0.0

    y_scr[...] += lax.dot_general(p, adj, (((0,), (0,)), ((), ())),
                                  preferred_element_type=jnp.float32)
    ssq_scr[0, 0] += jnp.sum(adj * adj)

    @pl.when(i == nb - 1)
    def _tail():
        _k6_tail(y_scr, p_scr, h_ref, ssq_scr, w2s_ref, w2n_ref, b2_ref,
                 fc1w_ref, fc1b_ref, fc2w_ref, fc2b_ref, z_ref, reg_ref)


def _k5_call(h, a1, a2, inv, ws_p, wn_p, b_p, adj, w2s_p, w2n_p, b2_p,
             fc1w_p, fc1b_p, fc2w, fc2b_p):
    return pl.pallas_call(
        _k5_body,
        grid=(N // BR,),
        in_specs=[
            pl.BlockSpec((N, W), lambda i: (0, 0)),
            pl.BlockSpec((BR, W), lambda i: (i, 0)),
            pl.BlockSpec((BR, W), lambda i: (i, 0)),
            pl.BlockSpec((BR, 1), lambda i: (i, 0)),
            pl.BlockSpec((W, W), lambda i: (0, 0)),
            pl.BlockSpec((W, W), lambda i: (0, 0)),
            pl.BlockSpec((1, W), lambda i: (0, 0)),
            pl.BlockSpec((BR, N), lambda i: (i, 0)),
            pl.BlockSpec((W, W), lambda i: (0, 0)),
            pl.BlockSpec((W, W), lambda i: (0, 0)),
            pl.BlockSpec((1, W), lambda i: (0, 0)),
            pl.BlockSpec((4, W, W), lambda i: (0, 0, 0)),
            pl.BlockSpec((1, W), lambda i: (0, 0)),
            pl.BlockSpec((W, 2), lambda i: (0, 0)),
            pl.BlockSpec((1, 2), lambda i: (0, 0)),
        ],
        out_specs=[
            pl.BlockSpec((1, 2), lambda i: (0, 0)),
            pl.BlockSpec((1, 1), lambda i: (0, 0), memory_space=pltpu.SMEM),
        ],
        out_shape=[
            jax.ShapeDtypeStruct((1, 2), jnp.float32),
            jax.ShapeDtypeStruct((1, 1), jnp.float32),
        ],
        scratch_shapes=[
            pltpu.VMEM((N, W), jnp.float32),
            pltpu.VMEM((W, N), jnp.float32),
            pltpu.SMEM((1, 1), jnp.float32),
        ],
        compiler_params=pltpu.CompilerParams(
            vmem_limit_bytes=100 * 1024 * 1024),
    )(h, a1, a2, inv, ws_p, wn_p, b_p, adj, w2s_p, w2n_p, b2_p, fc1w_p,
      fc1b_p, fc2w, fc2b_p)


def _k6_tail(y_ref, p_ref, h_ref, ssq_ref, w2s_ref, w2n_ref, b2_ref,
             fc1w_ref, fc1b_ref, fc2w_ref, fc2b_ref, z_ref, reg_ref):
    y = y_ref[...]
    p = p_ref[...]
    h = h_ref[...]

    adj1 = jnp.dot(y, p, preferred_element_type=jnp.float32)
    sts = lax.dot_general(p, p, (((0,), (0,)), ((), ())),
                          preferred_element_type=jnp.float32)
    h1 = lax.dot_general(p, h, (((0,), (0,)), ((), ())),
                         preferred_element_type=jnp.float32)

    rid = lax.broadcasted_iota(jnp.int32, (W, W), 0)
    cid = lax.broadcasted_iota(jnp.int32, (W, W), 1)
    tr = jnp.sum(jnp.where(rid == cid, adj1, 0.0))
    ssq = ssq_ref[0, 0]
    l1 = jnp.sqrt(jnp.maximum(ssq - 2.0 * tr + jnp.sum(sts * sts), 0.0))
    link1 = l1 / (N * N)
    ent1 = jnp.sum(-p * jnp.log(p + 1e-15)) / N
    reg1 = link1 + ent1

    m1 = jnp.sum(h1, axis=0, keepdims=True) / W
    s2 = (jnp.dot(h1, w2s_ref[...], preferred_element_type=jnp.float32)
          + jnp.dot(m1, w2n_ref[...], preferred_element_type=jnp.float32)
          + b2_ref[...])
    s2 = jnp.where(cid < 4, s2, -1e30)
    mx = jnp.max(s2, axis=-1, keepdims=True)
    e2 = jnp.exp(s2 - mx)
    p2 = e2 / jnp.sum(e2, axis=-1, keepdims=True)

    h2 = lax.dot_general(p2, h1, (((0,), (0,)), ((), ())),
                         preferred_element_type=jnp.float32)
    pp2 = lax.dot_general(p2, p2, (((1,), (1,)), ((), ())),
                          preferred_element_type=jnp.float32)
    dif = adj1 - pp2
    link2 = jnp.sqrt(jnp.sum(dif * dif)) / (W * W)
    ent2 = jnp.sum(-p2 * jnp.log(p2 + 1e-15)) / W
    reg2 = link2 + ent2

    z = jnp.zeros((1, W), jnp.float32)
    for r in range(4):
        row = jnp.sum(jnp.where(rid == r, h2, 0.0), axis=0, keepdims=True)
        z = z + jnp.dot(row, fc1w_ref[r], preferred_element_type=jnp.float32)
    z = jnp.maximum(z + fc1b_ref[...], 0.0)
    z_ref[...] = jnp.dot(z, fc2w_ref[...],
                         preferred_element_type=jnp.float32) + fc2b_ref[...]
    reg_ref[0, 0] = reg1 * 10.0 + reg2 * 0.1


def kernel(x, edge_index, edge_attr, adj, W1s, W1n, b1, Wp1s, Wp1n, bp1,
           Wp2s, Wp2n, bp2, fc1_w, fc1_b, fc2_w, fc2_b):
    f32 = jnp.float32

    ws_p = jnp.zeros((D, W), f32).at[:, :30].set(W1s)
    wn_p = jnp.zeros((D, W), f32).at[:, :30].set(W1n)
    b1_p = jnp.zeros((1, W), f32).at[0, :30].set(b1)
    wp1s_p = jnp.zeros((W, W), f32).at[:30, :].set(Wp1s)
    wp1n_p = jnp.zeros((W, W), f32).at[:30, :].set(Wp1n)
    bp1_p = bp1.reshape(1, W)
    wp2s_p = jnp.zeros((W, W), f32).at[:30, :4].set(Wp2s)
    wp2n_p = jnp.zeros((W, W), f32).at[:30, :4].set(Wp2n)
    bp2_p = jnp.zeros((1, W), f32).at[0, :4].set(bp2)
    fc1w_p = jnp.zeros((4, W, W), f32).at[:, :30, :].set(
        fc1_w.reshape(4, 30, W))
    fc1b_p = fc1_b.reshape(1, W)
    fc2b_p = fc2_b.reshape(1, 2)

    src3 = jnp.concatenate(
        [edge_index[0], jnp.zeros((EPAD - E,), jnp.int32)]).reshape(
            NW * NCHK, CH)
    dst3 = jnp.concatenate(
        [edge_index[1], jnp.full((EPAD - E,), N + 200, jnp.int32)]).reshape(
            NW * NCHK, CH)
    zeros32 = jnp.zeros((RPS, W), f32)

    cs, table1 = _k1_call(x, ws_p, wn_p)

    agg1 = _get_prop(W)(table1, src3, dst3, zeros32)
    a1a = lax.slice(agg1, (0, 0), (N, W))
    a1b = lax.slice(agg1, (NP, 0), (NP + N, W))

    h, invdeg = _k3_call(cs, a1a, a1b, b1_p)

    agg2 = _get_prop(W)(h, src3, dst3, zeros32)
    a2a = lax.slice(agg2, (0, 0), (N, W))
    a2b = lax.slice(agg2, (NP, 0), (NP + N, W))

    z, reg = _k5_call(h, a2a, a2b, invdeg, wp1s_p, wp1n_p, bp1_p, adj,
                      wp2s_p, wp2n_p, bp2_p, fc1w_p, fc1b_p, fc2_w, fc2b_p)
    return z, reg[0, 0]
